# Initial kernel scaffold; baseline (speedup 1.0000x reference)
import functools

import jax
import jax.numpy as jnp
from jax import lax
from jax.experimental import pallas as pl
from jax.experimental.pallas import tpu as pltpu
from jax.experimental.pallas import tpu_sc as plsc

_NUM_CORES = 2
_NUM_SUBCORES = 16
_NUM_WORKERS = _NUM_CORES * _NUM_SUBCORES
_LANES = 16

_CHUNK = 640


def _dequant_kernel(idx_hbm, w_hbm, s_hbm, out_hbm,
                    idx_v, w_v, s_v, out_v, sem_w, sem_s):
  n_total = idx_hbm.shape[0]
  per_worker = n_total // _NUM_WORKERS
  n_chunks = per_worker // _CHUNK

  wid = lax.axis_index("s") * _NUM_CORES + lax.axis_index("c")
  base_w = wid * per_worker

  iota = lax.iota(jnp.int32, _LANES)
  col = [iota * 4 + j for j in range(4)]

  def chunk_body(c, _):
    base = base_w + c * _CHUNK
    pltpu.sync_copy(idx_hbm.at[pl.ds(base, _CHUNK)], idx_v)
    cw = pltpu.async_copy(w_hbm.at[idx_v], w_v, sem_w)
    cs = pltpu.async_copy(s_hbm.at[idx_v], s_v, sem_s)
    cw.wait()
    cs.wait()

    def row_body(r, _):
      words = plsc.bitcast(w_v[r], jnp.int32)
      s = s_v[r]
      row_splat = jnp.full((_LANES,), r, jnp.int32)
      for j in range(4):
        b = (words << (24 - 8 * j)) >> 24 if j < 3 else words >> 24
        v = b.astype(jnp.float32) * s
        plsc.store_scatter(out_v, [row_splat, col[j]], v)
      return 0

    lax.fori_loop(0, _CHUNK, row_body, 0)
    pltpu.sync_copy(out_v, out_hbm.at[pl.ds(base, _CHUNK)])
    return 0

  lax.fori_loop(0, n_chunks, chunk_body, 0)


def kernel(input, weight_int8, scale):
  batch, hist = input.shape
  vocab, dim = weight_int8.shape
  n = batch * hist

  idx = input.reshape(n).astype(jnp.int32)
  scale_flat = scale.reshape(vocab)

  mesh = plsc.VectorSubcoreMesh(core_axis_name="c", subcore_axis_name="s")
  run = pl.kernel(
      _dequant_kernel,
      out_type=jax.ShapeDtypeStruct((n, dim), jnp.float32),
      mesh=mesh,
      scratch_types=[
          pltpu.VMEM((_CHUNK,), jnp.int32),
          pltpu.VMEM((_CHUNK, dim), jnp.int8),
          pltpu.VMEM((_CHUNK,), jnp.float32),
          pltpu.VMEM((_CHUNK, dim), jnp.float32),
          pltpu.SemaphoreType.DMA,
          pltpu.SemaphoreType.DMA,
      ],
  )
  out = run(idx, weight_int8, scale_flat)
  return out.reshape(batch, hist, dim)


# speedup vs baseline: 6.4402x; 6.4402x over previous
import functools

import jax
import jax.numpy as jnp
from jax import lax
from jax.experimental import pallas as pl
from jax.experimental.pallas import tpu as pltpu
from jax.experimental.pallas import tpu_sc as plsc

_NUM_CORES = 2
_NUM_SUBCORES = 16
_NUM_WORKERS = _NUM_CORES * _NUM_SUBCORES
_LANES = 16

_CHUNK = 640


def _dequant_kernel(idx_hbm, w_hbm, s_hbm, out_hbm,
                    idx_v, w_v, s_v, out_v, sem_w, sem_s):
  n_total = idx_hbm.shape[0]
  per_worker = n_total // _NUM_WORKERS
  n_chunks = per_worker // _CHUNK

  wid = lax.axis_index("s") * _NUM_CORES + lax.axis_index("c")
  base_w = wid * per_worker

  iota = lax.iota(jnp.int32, _LANES)
  col = [iota * 4 + j for j in range(4)]

  def chunk_body(c, _):
    base = base_w + c * _CHUNK
    pltpu.sync_copy(idx_hbm.at[pl.ds(base, _CHUNK)], idx_v)
    cw = pltpu.async_copy(w_hbm.at[idx_v], w_v, sem_w)
    cs = pltpu.async_copy(s_hbm.at[idx_v], s_v.at[pl.ds(0, _CHUNK)], sem_s)
    cw.wait()
    cs.wait()

    def row_body(r, _):
      words = w_v[r]
      row_splat = jnp.full((_LANES,), r, jnp.int32)
      s_vec = s_v[pl.ds(r, _LANES)]
      s = jnp.broadcast_to(s_vec[0], (_LANES,))
      for j in range(4):
        b = (words << (24 - 8 * j)) >> 24 if j < 3 else words >> 24
        v = b.astype(jnp.float32) * s
        plsc.store_scatter(out_v, [row_splat, col[j]], v)
      return 0

    lax.fori_loop(0, _CHUNK, row_body, 0)
    pltpu.sync_copy(out_v, out_hbm.at[pl.ds(base, _CHUNK)])
    return 0

  lax.fori_loop(0, n_chunks, chunk_body, 0)


def kernel(input, weight_int8, scale):
  batch, hist = input.shape
  vocab, dim = weight_int8.shape
  n = batch * hist

  idx = input.reshape(n).astype(jnp.int32)
  scale_flat = scale.reshape(vocab)
  w_words = lax.bitcast_convert_type(
      weight_int8.reshape(vocab, dim // 4, 4), jnp.int32)

  mesh = plsc.VectorSubcoreMesh(core_axis_name="c", subcore_axis_name="s")
  run = pl.kernel(
      _dequant_kernel,
      out_type=jax.ShapeDtypeStruct((n, dim), jnp.float32),
      mesh=mesh,
      compiler_params=pltpu.CompilerParams(
          needs_layout_passes=False, use_tc_tiling_on_sc=False),
      scratch_types=[
          pltpu.VMEM((_CHUNK,), jnp.int32),
          pltpu.VMEM((_CHUNK, dim // 4), jnp.int32),
          pltpu.VMEM((_CHUNK + _LANES,), jnp.float32),
          pltpu.VMEM((_CHUNK, dim), jnp.float32),
          pltpu.SemaphoreType.DMA,
          pltpu.SemaphoreType.DMA,
      ],
  )
  out = run(idx, w_words, scale_flat)
  return out.reshape(batch, hist, dim)


# speedup vs baseline: 8.3356x; 1.2943x over previous
import functools

import jax
import jax.numpy as jnp
from jax import lax
from jax.experimental import pallas as pl
from jax.experimental.pallas import tpu as pltpu
from jax.experimental.pallas import tpu_sc as plsc

_NUM_CORES = 2
_NUM_SUBCORES = 16
_NUM_WORKERS = _NUM_CORES * _NUM_SUBCORES
_LANES = 16

_CHUNK_B = 16


def _dequant_kernel(idx_hbm, w_hbm, s_hbm, out_hbm,
                    idx_v, w_v, s_v, out_v, sem_w, sem_s):
  batch, hist, dim = out_hbm.shape
  chunk = _CHUNK_B * hist
  b_per_worker = batch // _NUM_WORKERS
  n_chunks = b_per_worker // _CHUNK_B

  wid = lax.axis_index("s") * _NUM_CORES + lax.axis_index("c")

  iota = lax.iota(jnp.int32, _LANES)
  col = [iota * 4 + j for j in range(4)]

  def chunk_body(c, _):
    b_base = wid * b_per_worker + c * _CHUNK_B
    base = b_base * hist
    pltpu.sync_copy(idx_hbm.at[pl.ds(base, chunk)], idx_v)
    cw = pltpu.async_copy(w_hbm.at[idx_v], w_v, sem_w)
    cs = pltpu.async_copy(s_hbm.at[idx_v], s_v.at[pl.ds(0, chunk)], sem_s)
    cw.wait()
    cs.wait()

    def b_body(b, _):
      b_splat = jnp.full((_LANES,), b, jnp.int32)

      def l_body(l, _):
        r = b * hist + l
        packed = w_v[r, 0]
        words = plsc.bitcast(packed, jnp.int32)
        s_vec = s_v[pl.ds(r, _LANES)]
        s = jnp.broadcast_to(s_vec[0], (_LANES,))
        l_splat = jnp.full((_LANES,), l, jnp.int32)
        for j in range(4):
          v = (words << (24 - 8 * j)) >> 24 if j < 3 else words >> 24
          plsc.store_scatter(out_v, [b_splat, l_splat, col[j]],
                             v.astype(jnp.float32) * s)
        return 0

      lax.fori_loop(0, hist, l_body, 0)
      return 0

    lax.fori_loop(0, _CHUNK_B, b_body, 0)
    pltpu.sync_copy(out_v, out_hbm.at[pl.ds(b_base, _CHUNK_B)])
    return 0

  lax.fori_loop(0, n_chunks, chunk_body, 0)


def kernel(input, weight_int8, scale):
  batch, hist = input.shape
  vocab, dim = weight_int8.shape
  n = batch * hist
  chunk = _CHUNK_B * hist

  idx = input.reshape(n).astype(jnp.int32)
  scale_flat = scale.reshape(vocab)
  w_packed = weight_int8.reshape(vocab, 1, dim)

  mesh = plsc.VectorSubcoreMesh(core_axis_name="c", subcore_axis_name="s")
  run = pl.kernel(
      _dequant_kernel,
      out_type=jax.ShapeDtypeStruct((batch, hist, dim), jnp.float32),
      mesh=mesh,
      compiler_params=pltpu.CompilerParams(
          needs_layout_passes=False, use_tc_tiling_on_sc=False),
      scratch_types=[
          pltpu.VMEM((chunk,), jnp.int32),
          pltpu.VMEM((chunk, 1, dim), jnp.int8),
          pltpu.VMEM((chunk + _LANES,), jnp.float32),
          pltpu.VMEM((_CHUNK_B, hist, dim), jnp.float32),
          pltpu.SemaphoreType.DMA,
          pltpu.SemaphoreType.DMA,
      ],
  )
  return run(idx, w_packed, scale_flat)
